# baseline (speedup 1.0000x reference)
import functools

import jax
import jax.numpy as jnp
from jax import lax
from jax.experimental import pallas as pl
from jax.experimental.pallas import tpu as pltpu
from jax.experimental.pallas import tpu_sc as plsc

H = 512
W = 512
LANES = 16
ROWS_PER_BLOCK = 64
BLOCK = ROWS_PER_BLOCK * W
NC = 1
NS = 16
NW = NC * NS


def _lane(vec, j):
    return vec[j]


def _span_dmas(src_ref, out_ref, s, e, sem, do_start):
    n = e - s
    big = n >= ROWS_PER_BLOCK
    k_chunk = jnp.where(big, (n + ROWS_PER_BLOCK - 1) // ROWS_PER_BLOCK, 0)
    k_row = jnp.where(big, 0, jnp.maximum(n, 0))

    def chunk_body(i, carry):
        start = jnp.minimum(s + i * ROWS_PER_BLOCK, e - ROWS_PER_BLOCK)
        cp = pltpu.make_async_copy(src_ref.at[pl.ds(0, BLOCK)],
                                   out_ref.at[pl.ds(start * W, BLOCK)], sem)
        if do_start:
            cp.start()
        else:
            cp.wait()
        return carry

    def row_body(i, carry):
        cp = pltpu.make_async_copy(src_ref.at[pl.ds(0, W)],
                                   out_ref.at[pl.ds((s + i) * W, W)], sem)
        if do_start:
            cp.start()
        else:
            cp.wait()
        return carry

    lax.fori_loop(0, k_chunk, chunk_body, 0)
    lax.fori_loop(0, k_row, row_body, 0)


def _sc_kernel(bounds_hbm, out_hbm, pat_refs, zero_ref, bvec_ref, sem):
    wid = lax.axis_index("s") * NC + lax.axis_index("c")

    zv = jnp.zeros((LANES,), jnp.float32)

    def zrow(r, carry):
        for c in range(W // LANES):
            zero_ref[pl.ds(r * W + c * LANES, LANES)] = zv
        return carry

    lax.fori_loop(0, ROWS_PER_BLOCK, zrow, 0)

    spans = []
    drained = 0
    for t in range(4):
        b = wid * 4 + t
        pat_ref = pat_refs[t % 2]
        if t >= 2:
            for ref, s0, s1 in spans[drained:]:
                _span_dmas(ref, out_hbm, s0, s1, sem, do_start=False)
            drained = len(spans)
        pltpu.sync_copy(bounds_hbm.at[b], bvec_ref)
        bv = bvec_ref[...]
        xs = _lane(bv, 0)
        xe = _lane(bv, 1)
        ys = _lane(bv, 2)
        ye = _lane(bv, 3)
        lo = jnp.clip(ys, 0, H)
        hi = jnp.clip(ye + 1, lo, H)

        def prow(r, carry):
            for c in range(W // LANES):
                ix = lax.broadcasted_iota(jnp.int32, (LANES,), 0) + c * LANES
                val = jnp.where((ix >= xs) & (ix <= xe),
                                jnp.float32(1.0), jnp.float32(0.0))
                pat_ref[pl.ds(r * W + c * LANES, LANES)] = val
            return carry

        lax.fori_loop(0, ROWS_PER_BLOCK, prow, 0)

        base = b * H
        spans += [(zero_ref, base, base + lo),
                  (pat_ref, base + lo, base + hi),
                  (zero_ref, base + hi, base + H)]
        for ref, s0, s1 in spans[-3:]:
            _span_dmas(ref, out_hbm, s0, s1, sem, do_start=True)

    for ref, s0, s1 in spans[drained:]:
        _span_dmas(ref, out_hbm, s0, s1, sem, do_start=False)


def kernel(motion):
    B = motion.shape[0]
    xs = jnp.ceil(jnp.float32(H // 4) + motion[:, 0]).astype(jnp.int32)
    xe = jnp.ceil(jnp.float32(3 * H // 4) + motion[:, 0]).astype(jnp.int32)
    ys = jnp.ceil(jnp.float32(W // 4) + motion[:, 1]).astype(jnp.int32)
    ye = jnp.ceil(jnp.float32(3 * W // 4) + motion[:, 1]).astype(jnp.int32)
    bounds = jnp.zeros((B, LANES), jnp.int32)
    bounds = bounds.at[:, 0].set(xs).at[:, 1].set(xe)
    bounds = bounds.at[:, 2].set(ys).at[:, 3].set(ye)

    run = functools.partial(
        pl.kernel,
        mesh=plsc.VectorSubcoreMesh(core_axis_name="c", subcore_axis_name="s", num_cores=1),
        out_type=jax.ShapeDtypeStruct((B * H * W,), jnp.float32),
        scratch_types=[
            (pltpu.VMEM((BLOCK,), jnp.float32),
             pltpu.VMEM((BLOCK,), jnp.float32)),
            pltpu.VMEM((BLOCK,), jnp.float32),
            pltpu.VMEM((LANES,), jnp.int32),
            pltpu.SemaphoreType.DMA,
        ],
    )(_sc_kernel)
    out = run(bounds)
    return out.reshape(B, H, W, 1)


# speedup vs baseline: 1.5773x; 1.5773x over previous
import functools

import jax
import jax.numpy as jnp
from jax import lax
from jax.experimental import pallas as pl
from jax.experimental.pallas import tpu as pltpu
from jax.experimental.pallas import tpu_sc as plsc

H = 512
W = 512
LANES = 16
ROWS_PER_BLOCK = 64
BLOCK = ROWS_PER_BLOCK * W
NC = 2
NS = 16
NW = NC * NS


def _ceil_i32(v):
    t = v.astype(jnp.int32)
    return t + jnp.where(v > t.astype(jnp.float32),
                         jnp.int32(1), jnp.int32(0))


def _span_dmas(src_ref, out_ref, s, e, sem, do_start):
    n = e - s
    big = n >= ROWS_PER_BLOCK
    k_chunk = jnp.where(big, (n + ROWS_PER_BLOCK - 1) // ROWS_PER_BLOCK, 0)
    k_row = jnp.where(big, 0, jnp.maximum(n, 0))

    def chunk_body(i, carry):
        start = jnp.minimum(s + i * ROWS_PER_BLOCK, e - ROWS_PER_BLOCK)
        cp = pltpu.make_async_copy(src_ref.at[pl.ds(0, BLOCK)],
                                   out_ref.at[pl.ds(start * W, BLOCK)], sem)
        if do_start:
            cp.start()
        else:
            cp.wait()
        return carry

    def row_body(i, carry):
        cp = pltpu.make_async_copy(src_ref.at[pl.ds(0, W)],
                                   out_ref.at[pl.ds((s + i) * W, W)], sem)
        if do_start:
            cp.start()
        else:
            cp.wait()
        return carry

    lax.fori_loop(0, k_chunk, chunk_body, 0)
    lax.fori_loop(0, k_row, row_body, 0)


def _sc_kernel(motion_hbm, out_hbm, pat0_ref, pat1_ref, zero_ref,
               mvec_ref, sem):
    wid = lax.axis_index("s") * NC + lax.axis_index("c")

    zv = jnp.zeros((LANES,), jnp.float32)

    def zrow(r, carry):
        for c in range(W // LANES):
            zero_ref[pl.ds(r * W + c * LANES, LANES)] = zv
        return carry

    lax.fori_loop(0, ROWS_PER_BLOCK, zrow, 0)

    spans = []
    for t, pat_ref in enumerate((pat0_ref, pat1_ref)):
        b = wid * 2 + t
        pltpu.sync_copy(motion_hbm.at[b], mvec_ref)
        mv = mvec_ref[...]
        m0 = mv[0]
        m1 = mv[1]
        xs_v = _ceil_i32(jnp.float32(H // 4) + m0)
        xe_v = _ceil_i32(jnp.float32(3 * H // 4) + m0)
        ys_v = _ceil_i32(jnp.float32(W // 4) + m1)
        ye_v = _ceil_i32(jnp.float32(3 * W // 4) + m1)
        lo = jnp.clip(ys_v, 0, H)
        hi = jnp.clip(ye_v + 1, lo, H)

        def prow(r, carry):
            for c in range(W // LANES):
                ix = lax.broadcasted_iota(jnp.int32, (LANES,), 0) + c * LANES
                val = jnp.where((ix >= xs_v) & (ix <= xe_v),
                                jnp.float32(1.0), jnp.float32(0.0))
                pat_ref[pl.ds(r * W + c * LANES, LANES)] = val
            return carry

        lax.fori_loop(0, ROWS_PER_BLOCK, prow, 0)

        base = b * H
        spans += [(zero_ref, base, base + lo),
                  (pat_ref, base + lo, base + hi),
                  (zero_ref, base + hi, base + H)]
        for ref, s0, s1 in spans[-3:]:
            _span_dmas(ref, out_hbm, s0, s1, sem, do_start=True)

    for ref, s0, s1 in spans:
        _span_dmas(ref, out_hbm, s0, s1, sem, do_start=False)


def kernel(motion):
    B = motion.shape[0]
    run = functools.partial(
        pl.kernel,
        mesh=plsc.VectorSubcoreMesh(core_axis_name="c", subcore_axis_name="s"),
        out_type=jax.ShapeDtypeStruct((B * H * W,), jnp.float32),
        scratch_types=[
            pltpu.VMEM((BLOCK,), jnp.float32),
            pltpu.VMEM((BLOCK,), jnp.float32),
            pltpu.VMEM((BLOCK,), jnp.float32),
            pltpu.VMEM((LANES,), jnp.float32),
            pltpu.SemaphoreType.DMA,
        ],
    )(_sc_kernel)
    out = run(jnp.pad(motion, ((0, 0), (0, LANES - 2))))
    return out.reshape(B, H, W, 1)
